# Initial kernel scaffold; baseline (speedup 1.0000x reference)
import jax
import jax.numpy as jnp
from jax.experimental import pallas as pl


def kernel(first_index, attribute):
    raise NotImplementedError("write your pallas kernel here")


# speedup vs baseline: 21.5196x; 21.5196x over previous
import functools

import jax
import jax.numpy as jnp
from jax import lax
from jax.experimental import pallas as pl
from jax.experimental.pallas import tpu as pltpu
from jax.experimental.pallas import tpu_sc as plsc

KNB = 16
LSEQ = 2048
DATT = 128
NBATCH = 4
RBLK = 256

_NC = 2
_NS = 16
_NW = _NC * _NS
_TOTAL_ROWS = NBATCH * LSEQ * KNB
_ROWS_PER_W = _TOTAL_ROWS // _NW
_CHUNK = 128
_NCH = _ROWS_PER_W // _CHUNK


def _topk_body(ci_ref, cj_ref, nbr_ref, dist_ref):
    ci = ci_ref[0]
    cj = cj_ref[0]
    b = pl.program_id(0)
    ad = jnp.abs(ci - cj)
    j = lax.broadcasted_iota(jnp.int32, ad.shape, 1)
    keys = (ad << 11) | j
    nbr_cols = []
    dist_cols = []
    for _ in range(KNB):
        m = jnp.min(keys, axis=1, keepdims=True)
        keys = jnp.where(keys == m, jnp.int32(0x7FFFFFFF), keys)
        nbr_cols.append((m & 0x7FF) + b * LSEQ)
        dist_cols.append((m >> 11).astype(jnp.float32))
    nbr_ref[0] = jnp.concatenate(nbr_cols, axis=1)
    dist_ref[0] = jnp.concatenate(dist_cols, axis=1)


_topk_call = pl.pallas_call(
    _topk_body,
    grid=(NBATCH, LSEQ // RBLK),
    in_specs=[
        pl.BlockSpec((1, RBLK, 1), lambda b, i: (b, i, 0)),
        pl.BlockSpec((1, 1, LSEQ), lambda b, i: (b, 0, 0)),
    ],
    out_specs=[
        pl.BlockSpec((1, RBLK, KNB), lambda b, i: (b, i, 0)),
        pl.BlockSpec((1, RBLK, KNB), lambda b, i: (b, i, 0)),
    ],
    out_shape=[
        jax.ShapeDtypeStruct((NBATCH, LSEQ, KNB), jnp.int32),
        jax.ShapeDtypeStruct((NBATCH, LSEQ, KNB), jnp.float32),
    ],
)


def _gather_body(table_hbm, idx_hbm, out_hbm, idx_v, rows_v, sem):
    wid = lax.axis_index("s") * _NC + lax.axis_index("c")
    base = wid * _ROWS_PER_W

    def chunk(ch, carry):
        pltpu.sync_copy(idx_hbm.at[wid, ch], idx_v)
        pltpu.async_copy(table_hbm.at[idx_v], rows_v, sem).wait()
        pltpu.sync_copy(rows_v, out_hbm.at[pl.ds(base + ch * _CHUNK, _CHUNK)])
        return carry

    lax.fori_loop(0, _NCH, chunk, 0)


@functools.cache
def _make_gather_call():
    return pl.kernel(
        _gather_body,
        out_type=jax.ShapeDtypeStruct((_TOTAL_ROWS, DATT), jnp.float32),
        mesh=plsc.VectorSubcoreMesh(
            core_axis_name="c", subcore_axis_name="s",
            num_cores=_NC, num_subcores=_NS,
        ),
        scratch_types=[
            pltpu.VMEM((_CHUNK,), jnp.int32),
            pltpu.VMEM((_CHUNK, DATT), jnp.float32),
            pltpu.SemaphoreType.DMA,
        ],
    )


@jax.jit
def kernel(first_index, attribute):
    B, L, _ = first_index.shape
    ci = first_index
    cj = first_index.reshape(B, 1, L)
    nbr, dist = _topk_call(ci, cj)
    table = attribute.reshape(B * L, DATT)
    idx = nbr.reshape(_NW, _NCH, _CHUNK)
    rows = _make_gather_call()(table, idx)
    nb_attr = rows.reshape(B, L, KNB, DATT)
    index_distance = dist.reshape(B, L, KNB, 1)
    return (index_distance, nb_attr)


# speedup vs baseline: 25.8463x; 1.2011x over previous
import functools

import jax
import jax.numpy as jnp
from jax import lax
from jax.experimental import pallas as pl
from jax.experimental.pallas import tpu as pltpu
from jax.experimental.pallas import tpu_sc as plsc

KNB = 16
LSEQ = 2048
DATT = 128
NBATCH = 4
RBLK = 256

_NC = 2
_NS = 16
_NW = _NC * _NS
_TOTAL_ROWS = NBATCH * LSEQ * KNB
_ROWS_PER_W = _TOTAL_ROWS // _NW
_CHUNK = 128
_NCH = _ROWS_PER_W // _CHUNK


def _topk_body(ci_ref, cj_ref, nbr_ref, dist_ref):
    ci = ci_ref[0]
    cj = cj_ref[0]
    b = pl.program_id(0)
    ad = jnp.abs(ci - cj)
    j = lax.broadcasted_iota(jnp.int32, ad.shape, 1)
    keys = (ad << 11) | j
    q = jnp.full((RBLK, 1), jnp.int32(-(2 ** 31)))
    nbr_cols = []
    dist_cols = []
    for _ in range(KNB):
        m = jnp.min(keys - q, axis=1, keepdims=True)
        mi = m + q
        q = mi + jnp.int32(-(2 ** 31) + 1)
        nbr_cols.append((mi & 0x7FF) + b * LSEQ)
        dist_cols.append((mi >> 11).astype(jnp.float32))
    nbr_ref[0] = jnp.concatenate(nbr_cols, axis=1)
    dist_ref[0] = jnp.concatenate(dist_cols, axis=1)


_topk_call = pl.pallas_call(
    _topk_body,
    grid=(NBATCH, LSEQ // RBLK),
    in_specs=[
        pl.BlockSpec((1, RBLK, 1), lambda b, i: (b, i, 0)),
        pl.BlockSpec((1, 1, LSEQ), lambda b, i: (b, 0, 0)),
    ],
    out_specs=[
        pl.BlockSpec((1, RBLK, KNB), lambda b, i: (b, i, 0)),
        pl.BlockSpec((1, RBLK, KNB), lambda b, i: (b, i, 0)),
    ],
    out_shape=[
        jax.ShapeDtypeStruct((NBATCH, LSEQ, KNB), jnp.int32),
        jax.ShapeDtypeStruct((NBATCH, LSEQ, KNB), jnp.float32),
    ],
)


def _gather_body(table_hbm, idx_hbm, out_hbm, idx_all, rows0, rows1,
                 gsem0, gsem1, ssem0, ssem1):
    wid = lax.axis_index("s") * _NC + lax.axis_index("c")
    base = wid * _ROWS_PER_W
    pltpu.sync_copy(idx_hbm.at[wid], idx_all)

    def gather(ch, rows, sem):
        pltpu.make_async_copy(table_hbm.at[idx_all.at[ch]], rows, sem).start()

    def wait_gather(rows, sem):
        pltpu.make_async_copy(table_hbm.at[idx_all.at[0]], rows, sem).wait()

    def store(ch, rows, sem):
        dst = out_hbm.at[pl.ds(base + ch * _CHUNK, _CHUNK)]
        pltpu.make_async_copy(rows, dst, sem).start()

    def wait_store(rows, sem):
        dst = out_hbm.at[pl.ds(base, _CHUNK)]
        pltpu.make_async_copy(rows, dst, sem).wait()

    gather(0, rows0, gsem0)
    gather(1, rows1, gsem1)

    def step(t, carry):
        a = 2 * t
        wait_gather(rows0, gsem0)
        store(a, rows0, ssem0)
        wait_gather(rows1, gsem1)
        store(a + 1, rows1, ssem1)

        @pl.when(t < _NCH // 2 - 1)
        def _():
            wait_store(rows0, ssem0)
            gather(a + 2, rows0, gsem0)
            wait_store(rows1, ssem1)
            gather(a + 3, rows1, gsem1)

        return carry

    lax.fori_loop(0, _NCH // 2, step, 0)
    wait_store(rows0, ssem0)
    wait_store(rows1, ssem1)


@functools.cache
def _make_gather_call():
    return pl.kernel(
        _gather_body,
        out_type=jax.ShapeDtypeStruct((_TOTAL_ROWS, DATT), jnp.float32),
        mesh=plsc.VectorSubcoreMesh(
            core_axis_name="c", subcore_axis_name="s",
            num_cores=_NC, num_subcores=_NS,
        ),
        scratch_types=[
            pltpu.VMEM((_NCH, _CHUNK), jnp.int32),
            pltpu.VMEM((_CHUNK, DATT), jnp.float32),
            pltpu.VMEM((_CHUNK, DATT), jnp.float32),
            pltpu.SemaphoreType.DMA,
            pltpu.SemaphoreType.DMA,
            pltpu.SemaphoreType.DMA,
            pltpu.SemaphoreType.DMA,
        ],
    )


@jax.jit
def kernel(first_index, attribute):
    B, L, _ = first_index.shape
    ci = first_index
    cj = first_index.reshape(B, 1, L)
    nbr, dist = _topk_call(ci, cj)
    table = attribute.reshape(B * L, DATT)
    idx = nbr.reshape(_NW, _NCH, _CHUNK)
    rows = _make_gather_call()(table, idx)
    nb_attr = rows.reshape(B, L, KNB, DATT)
    index_distance = dist.reshape(B, L, KNB, 1)
    return (index_distance, nb_attr)
